# baseline (speedup 1.0000x reference)
import functools
import jax
import jax.numpy as jnp
from jax import lax
from jax.experimental import pallas as pl
from jax.experimental.pallas import tpu as pltpu
from jax.experimental.pallas import tpu_sc as plsc

B, L, D = 16, 2048, 1024


CH = 128
NCH = L // CH


NW = 32
CHS = 32
MAXCH = 16
NBUF = 2
NSL = D // 16
NGRP = 4
GSL = NSL // NGRP

TC_FRAC = 0.995


def _tc_body(len2_ref, in_hbm, out_ref, buf, sem):
    i = pl.program_id(0)
    n_loop = len2_ref[0, i]
    n_div = len2_ref[1, i]
    slot = lax.rem(i, 2)

    def mk(idx, sl, k):
        return pltpu.make_async_copy(
            in_hbm.at[idx, pl.ds(0, k * CH), :],
            buf.at[sl, pl.ds(0, k * CH), :],
            sem.at[sl],
        )

    def issue(idx, sl):
        kk = lax.div(len2_ref[0, idx] - 1, CH)
        lax.switch(kk, [lambda k=k: mk(idx, sl, k + 1).start()
                        for k in range(NCH)])

    def wait(idx, sl):
        kk = lax.div(len2_ref[0, idx] - 1, CH)
        lax.switch(kk, [lambda k=k: mk(idx, sl, k + 1).wait()
                        for k in range(NCH)])

    @pl.when(i == 0)
    def _():
        issue(0, 0)

    @pl.when(i + 1 < B)
    def _():
        issue(i + 1, lax.rem(i + 1, 2))

    wait(i, slot)

    nch = lax.div(n_loop - 1, CH) + 1

    def step(c, acc):
        rv = n_loop - c * CH

        def full_sum(_):
            return jnp.sum(buf[slot, pl.ds(c * CH, CH), :], axis=0)

        def masked_sum(_):
            row_id = lax.broadcasted_iota(jnp.int32, (CH, 1), 0)
            w = (row_id < rv).astype(jnp.float32)
            return jnp.sum(buf[slot, pl.ds(c * CH, CH), :] * w, axis=0)

        return acc + lax.cond(rv >= CH, full_sum, masked_sum, 0)

    acc = lax.fori_loop(0, nch, step, jnp.zeros((D,), jnp.float32))
    out_ref[i, :] = acc / n_div.astype(jnp.float32)


def _tc_partial(input, n_loop, n_div):
    len2 = jnp.stack([n_loop, n_div])
    grid_spec = pltpu.PrefetchScalarGridSpec(
        num_scalar_prefetch=1,
        grid=(B,),
        in_specs=[pl.BlockSpec(memory_space=pl.ANY)],
        out_specs=pl.BlockSpec((B, D), lambda i, len_r: (0, 0)),
        scratch_shapes=[
            pltpu.VMEM((2, L, D), jnp.float32),
            pltpu.SemaphoreType.DMA((2,)),
        ],
    )
    return pl.pallas_call(
        _tc_body,
        grid_spec=grid_spec,
        out_shape=jax.ShapeDtypeStruct((B, D), jnp.float32),
    )(len2, input)


def _sc_partial(x2, meta, mrow, invn):
    mesh = plsc.VectorSubcoreMesh(core_axis_name="c", subcore_axis_name="s")

    @functools.partial(
        pl.kernel,
        mesh=mesh,
        out_type=jax.ShapeDtypeStruct((B, 8, 128), jnp.float32),
        scratch_types=[
            pltpu.VMEM((NBUF, CHS, D), jnp.float32),
            pltpu.VMEM((8, 128), jnp.float32),
            pltpu.VMEM((MAXCH, 16), jnp.int32),
            pltpu.VMEM((16,), jnp.int32),
            pltpu.VMEM((B, 16), jnp.float32),
            pltpu.VMEM((8, 128), jnp.float32),
            pltpu.VMEM((8, 128), jnp.float32),
            pltpu.VMEM_SHARED((16, 8, 128), jnp.float32),
            pltpu.SemaphoreType.DMA((NBUF,)),
        ],
    )
    def k(x_hbm, meta_hbm, mrow_hbm, invn_hbm, out_hbm,
          buf, acc, meta_v, m_v, invn_v, tmpa, tmpb, shared, sem):
        c = lax.axis_index("c")
        s = lax.axis_index("s")
        wid = c * 16 + s
        b1 = lax.div(wid, 2)

        pltpu.sync_copy(meta_hbm.at[wid], meta_v)
        pltpu.sync_copy(mrow_hbm.at[wid], m_v)
        pltpu.sync_copy(invn_hbm, invn_v)
        mw = m_v[...][0]

        zero = jnp.zeros((16,), jnp.float32)
        for t in range(NSL):
            acc[t // 8, pl.ds((t % 8) * 16, 16)] = zero

        def chunk_meta(j):
            v = meta_v[j, :]
            return v[0], v[1], v[2]

        def cp(j, slot):
            g0 = pl.multiple_of(chunk_meta(j)[0], 8)
            return pltpu.make_async_copy(
                x_hbm.at[pl.ds(g0, CHS), :],
                buf.at[slot],
                sem.at[slot],
            )

        for t in range(NBUF - 1):
            @pl.when(t < mw)
            def _():
                cp(t, t).start()

        iv = invn_v[b1, :]

        def chunk_step(j, carry):
            slot = lax.rem(j, NBUF)
            jn = j + NBUF - 1

            @pl.when(jn < mw)
            def _():
                cp(jn, lax.rem(jn, NBUF)).start()

            cp(j, slot).wait()
            _, lo, hi = chunk_meta(j)

            for g in range(NGRP):
                def row_step(r, part):
                    return tuple(
                        part[t] + buf[slot, r, pl.ds((g * GSL + t) * 16, 16)]
                        for t in range(GSL))

                part = lax.fori_loop(
                    lo, hi, row_step,
                    tuple(jnp.zeros((16,), jnp.float32) for _ in range(GSL)))
                for t in range(GSL):
                    tt = g * GSL + t
                    sl = pl.ds((tt % 8) * 16, 16)
                    acc[tt // 8, sl] = acc[tt // 8, sl] + part[t] * iv
            return carry

        lax.fori_loop(0, mw, chunk_step, 0)

        pltpu.sync_copy(acc, shared.at[s])
        plsc.subcore_barrier()

        @pl.when(s < 8)
        def _():
            bout = c * 8 + s
            pltpu.sync_copy(shared.at[2 * s], tmpa)
            pltpu.sync_copy(shared.at[2 * s + 1], tmpb)
            for t in range(NSL):
                sl = pl.ds((t % 8) * 16, 16)
                tmpa[t // 8, sl] = tmpa[t // 8, sl] + tmpb[t // 8, sl]
            pltpu.sync_copy(tmpa, out_hbm.at[bout])

    return k(x2, meta, mrow, invn)


def _sc_tail_partials(input, n, n_tc):
    x2 = input.reshape(B * L, D)
    m = n - n_tc
    q2 = 8 * ((m + 15) // 16)
    wi = jnp.arange(NW, dtype=jnp.int32)
    bat1 = wi // 2
    h = wi % 2
    ntb = n_tc[bat1]
    mb = m[bat1]
    q2b = q2[bat1]
    ws = ntb + jnp.minimum(h * q2b, mb)
    we = ntb + jnp.minimum((h + 1) * q2b, mb)

    k = jnp.arange(MAXCH, dtype=jnp.int32)
    cs = ws[:, None] + CHS * k[None, :]
    ce = jnp.minimum(we[:, None], cs + CHS)
    valid = ce > cs
    dma0 = jnp.minimum(cs, L - CHS)
    g0 = (bat1 * L)[:, None] + dma0
    lo = cs - dma0
    hi = jnp.where(valid, ce - dma0, lo)
    bat = jnp.broadcast_to(bat1[:, None], (NW, MAXCH))
    mcnt = valid.sum(axis=1).astype(jnp.int32)

    meta = jnp.zeros((NW, MAXCH, 16), jnp.int32)
    meta = meta.at[:, :, 0].set(g0)
    meta = meta.at[:, :, 1].set(lo)
    meta = meta.at[:, :, 2].set(hi)
    meta = meta.at[:, :, 3].set(bat)
    mrow = jnp.zeros((NW, 16), jnp.int32).at[:, 0].set(mcnt)
    invn = jnp.broadcast_to(
        (1.0 / n.astype(jnp.float32))[:, None], (B, 16))

    return _sc_partial(x2, meta, mrow, invn).reshape(B, D)


def kernel(input, length):
    n = length.astype(jnp.int32)
    n_tc = jnp.clip(8 * jnp.int32(jnp.round(n * (TC_FRAC / 8.0))), 8, n)
    tc = _tc_partial(input, n_tc, n)
    sc = _sc_tail_partials(input, n, n_tc)
    return tc + sc


# speedup vs baseline: 2.2160x; 2.2160x over previous
import jax
import jax.numpy as jnp
from jax import lax
from jax.experimental import pallas as pl
from jax.experimental.pallas import tpu as pltpu

B, L, D = 16, 2048, 1024
CH = 128
NCH = L // CH


def _body(len_ref, in_hbm, out_ref, buf, sem):
    i = pl.program_id(0)
    n = len_ref[i]
    slot = lax.rem(i, 2)

    def mk(idx, sl, k):
        return pltpu.make_async_copy(
            in_hbm.at[idx, pl.ds(0, k * CH), :],
            buf.at[sl, pl.ds(0, k * CH), :],
            sem.at[sl],
        )

    def issue(idx, sl):
        kk = lax.div(len_ref[idx] - 1, CH)
        lax.switch(kk, [lambda k=k: mk(idx, sl, k + 1).start()
                        for k in range(NCH)])

    def wait(idx, sl):
        kk = lax.div(len_ref[idx] - 1, CH)
        lax.switch(kk, [lambda k=k: mk(idx, sl, k + 1).wait()
                        for k in range(NCH)])

    @pl.when(i == 0)
    def _():
        issue(0, 0)

    @pl.when(i + 1 < B)
    def _():
        issue(i + 1, lax.rem(i + 1, 2))

    wait(i, slot)

    nch = lax.div(n - 1, CH) + 1

    def step(c, acc):
        rv = n - c * CH

        def full_sum(_):
            return jnp.sum(buf[slot, pl.ds(c * CH, CH), :], axis=0)

        def masked_sum(_):
            row_id = lax.broadcasted_iota(jnp.int32, (CH, 1), 0)
            w = (row_id < rv).astype(jnp.float32)
            return jnp.sum(buf[slot, pl.ds(c * CH, CH), :] * w, axis=0)

        return acc + lax.cond(rv >= CH, full_sum, masked_sum, 0)

    acc = lax.fori_loop(0, nch, step, jnp.zeros((D,), jnp.float32))
    out_ref[i, :] = acc / n.astype(jnp.float32)


def kernel(input, length):
    n = length.astype(jnp.int32)
    grid_spec = pltpu.PrefetchScalarGridSpec(
        num_scalar_prefetch=1,
        grid=(B,),
        in_specs=[pl.BlockSpec(memory_space=pl.ANY)],
        out_specs=pl.BlockSpec((B, D), lambda i, len_r: (0, 0)),
        scratch_shapes=[
            pltpu.VMEM((2, L, D), jnp.float32),
            pltpu.SemaphoreType.DMA((2,)),
        ],
    )
    return pl.pallas_call(
        _body,
        grid_spec=grid_spec,
        out_shape=jax.ShapeDtypeStruct((B, D), jnp.float32),
    )(n, input)
